# Initial kernel scaffold; baseline (speedup 1.0000x reference)
import jax
import jax.numpy as jnp
from jax.experimental import pallas as pl


def kernel(entity_emb, relation_emb, edge_index, edge_type):
    raise NotImplementedError("write your pallas kernel here")


# speedup vs baseline: 1.1147x; 1.1147x over previous
import functools

import jax
import jax.numpy as jnp
from jax import lax
from jax.experimental import pallas as pl
from jax.experimental.pallas import tpu as pltpu
from jax.experimental.pallas import tpu_sc as plsc

N_HOPS = 2

_NC = 2
_NS = 16
_NW = _NC * _NS
_B = 80


def _sc_compiler_params():
    import dataclasses
    cp = pltpu.CompilerParams()
    if "needs_layout_passes" in pltpu.CompilerParams.__dataclass_fields__:
        cp = dataclasses.replace(cp, needs_layout_passes=False)
    return cp


def _row_split(n):
    base = (n // _NS) // 8 * 8
    last = n - base * (_NS - 1)
    assert last % 8 == 0
    return base, last


def _score_body(nb, n, d, emb_hbm, rel_hbm, head_hbm, tail_hbm, ty_hbm,
                score_out, tilemax_out,
                rel_v, hidx_v, tidx_v, ty_v, sblk_v, maxtab_v,
                hrows_v, trows_v, sem):
    c = lax.axis_index("c")
    s = lax.axis_index("s")
    wid = c * _NS + s

    pltpu.sync_copy(rel_hbm, rel_v)
    pltpu.sync_copy(head_hbm.at[wid], hidx_v)
    pltpu.sync_copy(tail_hbm.at[wid], tidx_v)
    pltpu.sync_copy(ty_hbm.at[wid], ty_v)

    neg = jnp.full((16,), -3.0e38, jnp.float32)

    @pl.loop(0, n, step=16)
    def _(i):
        maxtab_v[pl.ds(i, 16)] = neg

    @pl.loop(0, nb)
    def _(blk):
        cp_h = pltpu.async_copy(emb_hbm.at[hidx_v.at[blk]], hrows_v, sem)
        cp_t = pltpu.async_copy(emb_hbm.at[tidx_v.at[blk]], trows_v, sem)
        cp_h.wait()
        cp_t.wait()

        @pl.loop(0, _B, step=16)
        def _(eg):
            eids = lax.iota(jnp.int32, 16) + eg
            blks = jnp.zeros((16,), jnp.int32) + blk
            tys = plsc.load_gather(ty_v, [blks, eids])

            def dstep(i, acc):
                dsp = jnp.zeros((16,), jnp.int32) + i
                h = plsc.load_gather(hrows_v, [eids, dsp])
                t = plsc.load_gather(trows_v, [eids, dsp])
                r = plsc.load_gather(rel_v, [tys, dsp])
                return acc + h * t * r

            acc = lax.fori_loop(0, d, dstep, jnp.zeros((16,), jnp.float32))
            sblk_v[blk, pl.ds(eg, 16)] = acc

            hh = plsc.load_gather(hidx_v, [blks, eids])

            def mcond(pend):
                return jnp.any(pend)

            def mbody(pend):
                cur = plsc.load_gather(maxtab_v, [hh])
                need = pend & (acc > cur)
                plsc.store_scatter(maxtab_v, [hh], acc, mask=need)
                cur2 = plsc.load_gather(maxtab_v, [hh])
                return need & (cur2 < acc)

            lax.while_loop(mcond, mbody, jnp.ones((16,), jnp.bool_))

    pltpu.sync_copy(sblk_v, score_out.at[wid])
    pltpu.sync_copy(maxtab_v, tilemax_out.at[pl.ds(wid * n, n)])


def _hop_body(nb, n, d, x_hbm, rel_hbm, pack_hbm, tilemax_hbm, zeros_hbm,
              part_out,
              rel_v, ibuf_v, segmax_v, m0_v, m1_v, rows_v, orows_v,
              acc_sh, smax_sh, sem):
    c = lax.axis_index("c")
    s = lax.axis_index("s")
    wid = c * _NS + s
    rbase, rlast = _row_split(n)

    pltpu.sync_copy(rel_hbm, rel_v)

    def _combine_rows(r0, cnt):
        pltpu.sync_copy(tilemax_hbm.at[pl.ds(r0, cnt)], m0_v.at[pl.ds(0, cnt)])

        @pl.loop(1, _NW)
        def _(j):
            pltpu.sync_copy(tilemax_hbm.at[pl.ds(j * n + r0, cnt)],
                            m1_v.at[pl.ds(0, cnt)])

            @pl.loop(0, cnt, step=16)
            def _(i):
                m0_v[pl.ds(i, 16)] = jnp.maximum(m0_v[pl.ds(i, 16)],
                                                 m1_v[pl.ds(i, 16)])

        pltpu.sync_copy(m0_v.at[pl.ds(0, cnt)], smax_sh.at[pl.ds(r0, cnt)])
        pltpu.sync_copy(zeros_hbm.at[pl.ds(r0, cnt)], acc_sh.at[pl.ds(r0, cnt)])

    @pl.when(s < _NS - 1)
    def _():
        _combine_rows(s * rbase, rbase)

    @pl.when(s == _NS - 1)
    def _():
        _combine_rows((_NS - 1) * rbase, rlast)

    plsc.subcore_barrier()
    pltpu.sync_copy(smax_sh, segmax_v)

    @pl.loop(0, nb)
    def _(blk):
        pltpu.sync_copy(pack_hbm.at[wid, blk], ibuf_v)
        pltpu.async_copy(x_hbm.at[ibuf_v.at[1]], rows_v, sem).wait()

        @pl.loop(0, _B, step=16)
        def _(eg):
            eids = lax.iota(jnp.int32, 16) + eg
            sc = plsc.bitcast(ibuf_v[3, pl.ds(eg, 16)], jnp.float32)
            hh = ibuf_v[0, pl.ds(eg, 16)]
            mx = plsc.load_gather(segmax_v, [hh])
            w = jnp.exp(sc - mx)
            tys = ibuf_v[2, pl.ds(eg, 16)]

            def dstep(i, carry):
                dsp = jnp.zeros((16,), jnp.int32) + i
                xv = plsc.load_gather(rows_v, [eids, dsp])
                rv = plsc.load_gather(rel_v, [tys, dsp])
                plsc.store_scatter(orows_v, [eids, dsp], xv * rv * w)
                return carry

            lax.fori_loop(0, d, dstep, 0)

        pltpu.sync_copy(orows_v, acc_sh.at[ibuf_v.at[0]], add=True)

    plsc.subcore_barrier()

    @pl.when(s < _NS - 1)
    def _():
        pltpu.sync_copy(acc_sh.at[pl.ds(s * rbase, rbase)],
                        part_out.at[c, pl.ds(s * rbase, rbase)])

    @pl.when(s == _NS - 1)
    def _():
        pltpu.sync_copy(acc_sh.at[pl.ds((_NS - 1) * rbase, rlast)],
                        part_out.at[c, pl.ds((_NS - 1) * rbase, rlast)])


def _combine_body(p_ref, res_ref, xout_ref, resout_ref):
    sblk = p_ref[0] + p_ref[1]
    nrm = jnp.sqrt(jnp.sum(sblk * sblk, axis=1, keepdims=True))
    x = sblk / jnp.maximum(nrm, 1e-12)
    xout_ref[...] = x
    resout_ref[...] = res_ref[...] + x


def kernel(entity_emb, relation_emb, edge_index, edge_type):
    n, d = entity_emb.shape
    e = edge_index.shape[1]
    nrel = relation_emb.shape[0]
    epw = e // _NW
    nb = epw // _B
    assert epw * _NW == e and nb * _B == epw

    head = edge_index[0].astype(jnp.int32)
    tail = edge_index[1].astype(jnp.int32)
    ty = edge_type.astype(jnp.int32) - 1
    head1, tail1, ty1 = (a.reshape(_NW, nb, _B) for a in (head, tail, ty))
    rel = relation_emb.astype(jnp.float32)
    zeros = jnp.zeros((n, d), jnp.float32)

    mesh = plsc.VectorSubcoreMesh(core_axis_name="c", subcore_axis_name="s")
    cp = _sc_compiler_params()

    score_kernel = pl.kernel(
        functools.partial(_score_body, nb, n, d),
        out_type=(jax.ShapeDtypeStruct((_NW, nb, _B), jnp.float32),
                  jax.ShapeDtypeStruct((_NW * n,), jnp.float32)),
        mesh=mesh,
        compiler_params=cp,
        scratch_types=[
            pltpu.VMEM((nrel, d), jnp.float32),
            pltpu.VMEM((nb, _B), jnp.int32),
            pltpu.VMEM((nb, _B), jnp.int32),
            pltpu.VMEM((nb, _B), jnp.int32),
            pltpu.VMEM((nb, _B), jnp.float32),
            pltpu.VMEM((n,), jnp.float32),
            pltpu.VMEM((_B, d), jnp.float32),
            pltpu.VMEM((_B, d), jnp.float32),
            pltpu.SemaphoreType.DMA,
        ],
    )

    rlast = _row_split(n)[1]
    hop_kernel = pl.kernel(
        functools.partial(_hop_body, nb, n, d),
        out_type=jax.ShapeDtypeStruct((_NC, n, d), jnp.float32),
        mesh=mesh,
        compiler_params=cp,
        scratch_types=[
            pltpu.VMEM((nrel, d), jnp.float32),
            pltpu.VMEM((4, _B), jnp.int32),
            pltpu.VMEM((n,), jnp.float32),
            pltpu.VMEM((rlast,), jnp.float32),
            pltpu.VMEM((rlast,), jnp.float32),
            pltpu.VMEM((_B, d), jnp.float32),
            pltpu.VMEM((_B, d), jnp.float32),
            pltpu.VMEM_SHARED((n, d), jnp.float32),
            pltpu.VMEM_SHARED((n,), jnp.float32),
            pltpu.SemaphoreType.DMA,
        ],
    )

    rows_blk = 1000
    combine = pl.pallas_call(
        _combine_body,
        grid=(n // rows_blk,),
        in_specs=[
            pl.BlockSpec((_NC, rows_blk, d), lambda i: (0, i, 0)),
            pl.BlockSpec((rows_blk, d), lambda i: (i, 0)),
        ],
        out_specs=[
            pl.BlockSpec((rows_blk, d), lambda i: (i, 0)),
            pl.BlockSpec((rows_blk, d), lambda i: (i, 0)),
        ],
        out_shape=(jax.ShapeDtypeStruct((n, d), jnp.float32),
                   jax.ShapeDtypeStruct((n, d), jnp.float32)),
    )

    score, tilemax = score_kernel(entity_emb, rel, head1, tail1, ty1)
    pack = jnp.stack(
        [head1, tail1, ty1,
         jax.lax.bitcast_convert_type(score, jnp.int32)], axis=2)
    res = entity_emb
    x = entity_emb
    for _ in range(N_HOPS):
        part = hop_kernel(x, rel, pack, tilemax, zeros)
        x, res = combine(part, res)
    return res


# speedup vs baseline: 1.1589x; 1.0397x over previous
import functools

import jax
import jax.numpy as jnp
from jax import lax
from jax.experimental import pallas as pl
from jax.experimental.pallas import tpu as pltpu
from jax.experimental.pallas import tpu_sc as plsc

N_HOPS = 2

_NC = 2
_NS = 16
_NW = _NC * _NS
_B = 80


def _sc_compiler_params():
    import dataclasses
    cp = pltpu.CompilerParams()
    if "needs_layout_passes" in pltpu.CompilerParams.__dataclass_fields__:
        cp = dataclasses.replace(cp, needs_layout_passes=False)
    return cp


def _row_split(n):
    base = (n // _NS) // 8 * 8
    last = n - base * (_NS - 1)
    assert last % 8 == 0
    return base, last


def _score_body(nb, n, d, emb_hbm, rel_hbm, head_hbm, tail_hbm, ty_hbm,
                score_out, tilemax_out,
                rel_v, hidx_v, tidx_v, ty_v, sblk_v, maxtab_v,
                hrows_v, trows_v, sem):
    c = lax.axis_index("c")
    s = lax.axis_index("s")
    wid = c * _NS + s

    pltpu.sync_copy(rel_hbm, rel_v)
    pltpu.sync_copy(head_hbm.at[wid], hidx_v)
    pltpu.sync_copy(tail_hbm.at[wid], tidx_v)
    pltpu.sync_copy(ty_hbm.at[wid], ty_v)

    neg = jnp.full((16,), -3.0e38, jnp.float32)

    @pl.loop(0, n, step=16)
    def _(i):
        maxtab_v[pl.ds(i, 16)] = neg

    @pl.loop(0, nb)
    def _(blk):
        cp_h = pltpu.async_copy(emb_hbm.at[hidx_v.at[blk]], hrows_v, sem)
        cp_t = pltpu.async_copy(emb_hbm.at[tidx_v.at[blk]], trows_v, sem)
        cp_h.wait()
        cp_t.wait()

        @pl.loop(0, _B, step=16)
        def _(eg):
            eids = lax.iota(jnp.int32, 16) + eg
            blks = jnp.zeros((16,), jnp.int32) + blk
            tys = plsc.load_gather(ty_v, [blks, eids])

            acc = jnp.zeros((16,), jnp.float32)
            for i in range(d):
                dsp = jnp.zeros((16,), jnp.int32) + i
                h = plsc.load_gather(hrows_v, [eids, dsp])
                t = plsc.load_gather(trows_v, [eids, dsp])
                r = plsc.load_gather(rel_v, [tys, dsp])
                acc = acc + h * t * r
            sblk_v[blk, pl.ds(eg, 16)] = acc

            hh = plsc.load_gather(hidx_v, [blks, eids])

            def mcond(pend):
                return jnp.any(pend)

            def mbody(pend):
                cur = plsc.load_gather(maxtab_v, [hh])
                need = pend & (acc > cur)
                plsc.store_scatter(maxtab_v, [hh], acc, mask=need)
                cur2 = plsc.load_gather(maxtab_v, [hh])
                return need & (cur2 < acc)

            lax.while_loop(mcond, mbody, jnp.ones((16,), jnp.bool_))

    pltpu.sync_copy(sblk_v, score_out.at[wid])
    pltpu.sync_copy(maxtab_v, tilemax_out.at[pl.ds(wid * n, n)])


def _hop_body(nb, n, d, x_hbm, rel_hbm, pack_hbm, tilemax_hbm, zeros_hbm,
              part_out,
              rel_v, ibuf_v, segmax_v, m0_v, m1_v, rows_v, orows_v,
              acc_sh, smax_sh, sem):
    c = lax.axis_index("c")
    s = lax.axis_index("s")
    wid = c * _NS + s
    rbase, rlast = _row_split(n)

    pltpu.sync_copy(rel_hbm, rel_v)

    def _combine_rows(r0, cnt):
        pltpu.sync_copy(tilemax_hbm.at[pl.ds(r0, cnt)], m0_v.at[pl.ds(0, cnt)])

        @pl.loop(1, _NW)
        def _(j):
            pltpu.sync_copy(tilemax_hbm.at[pl.ds(j * n + r0, cnt)],
                            m1_v.at[pl.ds(0, cnt)])

            @pl.loop(0, cnt, step=16)
            def _(i):
                m0_v[pl.ds(i, 16)] = jnp.maximum(m0_v[pl.ds(i, 16)],
                                                 m1_v[pl.ds(i, 16)])

        pltpu.sync_copy(m0_v.at[pl.ds(0, cnt)], smax_sh.at[pl.ds(r0, cnt)])
        pltpu.sync_copy(zeros_hbm.at[pl.ds(r0, cnt)], acc_sh.at[pl.ds(r0, cnt)])

    @pl.when(s < _NS - 1)
    def _():
        _combine_rows(s * rbase, rbase)

    @pl.when(s == _NS - 1)
    def _():
        _combine_rows((_NS - 1) * rbase, rlast)

    plsc.subcore_barrier()
    pltpu.sync_copy(smax_sh, segmax_v)

    @pl.loop(0, nb)
    def _(blk):
        pltpu.sync_copy(pack_hbm.at[wid, blk], ibuf_v)
        pltpu.async_copy(x_hbm.at[ibuf_v.at[1]], rows_v, sem).wait()

        @pl.loop(0, _B, step=16)
        def _(eg):
            eids = lax.iota(jnp.int32, 16) + eg
            sc = plsc.bitcast(ibuf_v[3, pl.ds(eg, 16)], jnp.float32)
            hh = ibuf_v[0, pl.ds(eg, 16)]
            mx = plsc.load_gather(segmax_v, [hh])
            w = jnp.exp(sc - mx)
            tys = ibuf_v[2, pl.ds(eg, 16)]

            for i in range(d):
                dsp = jnp.zeros((16,), jnp.int32) + i
                xv = plsc.load_gather(rows_v, [eids, dsp])
                rv = plsc.load_gather(rel_v, [tys, dsp])
                plsc.store_scatter(orows_v, [eids, dsp], xv * rv * w)

        pltpu.sync_copy(orows_v, acc_sh.at[ibuf_v.at[0]], add=True)

    plsc.subcore_barrier()

    @pl.when(s < _NS - 1)
    def _():
        pltpu.sync_copy(acc_sh.at[pl.ds(s * rbase, rbase)],
                        part_out.at[c, pl.ds(s * rbase, rbase)])

    @pl.when(s == _NS - 1)
    def _():
        pltpu.sync_copy(acc_sh.at[pl.ds((_NS - 1) * rbase, rlast)],
                        part_out.at[c, pl.ds((_NS - 1) * rbase, rlast)])


def _combine_body(p_ref, res_ref, xout_ref, resout_ref):
    sblk = p_ref[0] + p_ref[1]
    nrm = jnp.sqrt(jnp.sum(sblk * sblk, axis=1, keepdims=True))
    x = sblk / jnp.maximum(nrm, 1e-12)
    xout_ref[...] = x
    resout_ref[...] = res_ref[...] + x


def kernel(entity_emb, relation_emb, edge_index, edge_type):
    n, d = entity_emb.shape
    e = edge_index.shape[1]
    nrel = relation_emb.shape[0]
    epw = e // _NW
    nb = epw // _B
    assert epw * _NW == e and nb * _B == epw

    head = edge_index[0].astype(jnp.int32)
    tail = edge_index[1].astype(jnp.int32)
    ty = edge_type.astype(jnp.int32) - 1
    head1, tail1, ty1 = (a.reshape(_NW, nb, _B) for a in (head, tail, ty))
    rel = relation_emb.astype(jnp.float32)
    zeros = jnp.zeros((n, d), jnp.float32)

    mesh = plsc.VectorSubcoreMesh(core_axis_name="c", subcore_axis_name="s")
    cp = _sc_compiler_params()

    score_kernel = pl.kernel(
        functools.partial(_score_body, nb, n, d),
        out_type=(jax.ShapeDtypeStruct((_NW, nb, _B), jnp.float32),
                  jax.ShapeDtypeStruct((_NW * n,), jnp.float32)),
        mesh=mesh,
        compiler_params=cp,
        scratch_types=[
            pltpu.VMEM((nrel, d), jnp.float32),
            pltpu.VMEM((nb, _B), jnp.int32),
            pltpu.VMEM((nb, _B), jnp.int32),
            pltpu.VMEM((nb, _B), jnp.int32),
            pltpu.VMEM((nb, _B), jnp.float32),
            pltpu.VMEM((n,), jnp.float32),
            pltpu.VMEM((_B, d), jnp.float32),
            pltpu.VMEM((_B, d), jnp.float32),
            pltpu.SemaphoreType.DMA,
        ],
    )

    rlast = _row_split(n)[1]
    hop_kernel = pl.kernel(
        functools.partial(_hop_body, nb, n, d),
        out_type=jax.ShapeDtypeStruct((_NC, n, d), jnp.float32),
        mesh=mesh,
        compiler_params=cp,
        scratch_types=[
            pltpu.VMEM((nrel, d), jnp.float32),
            pltpu.VMEM((4, _B), jnp.int32),
            pltpu.VMEM((n,), jnp.float32),
            pltpu.VMEM((rlast,), jnp.float32),
            pltpu.VMEM((rlast,), jnp.float32),
            pltpu.VMEM((_B, d), jnp.float32),
            pltpu.VMEM((_B, d), jnp.float32),
            pltpu.VMEM_SHARED((n, d), jnp.float32),
            pltpu.VMEM_SHARED((n,), jnp.float32),
            pltpu.SemaphoreType.DMA,
        ],
    )

    rows_blk = 1000
    combine = pl.pallas_call(
        _combine_body,
        grid=(n // rows_blk,),
        in_specs=[
            pl.BlockSpec((_NC, rows_blk, d), lambda i: (0, i, 0)),
            pl.BlockSpec((rows_blk, d), lambda i: (i, 0)),
        ],
        out_specs=[
            pl.BlockSpec((rows_blk, d), lambda i: (i, 0)),
            pl.BlockSpec((rows_blk, d), lambda i: (i, 0)),
        ],
        out_shape=(jax.ShapeDtypeStruct((n, d), jnp.float32),
                   jax.ShapeDtypeStruct((n, d), jnp.float32)),
    )

    score, tilemax = score_kernel(entity_emb, rel, head1, tail1, ty1)
    pack = jnp.stack(
        [head1, tail1, ty1,
         jax.lax.bitcast_convert_type(score, jnp.int32)], axis=2)
    res = entity_emb
    x = entity_emb
    for _ in range(N_HOPS):
        part = hop_kernel(x, rel, pack, tilemax, zeros)
        x, res = combine(part, res)
    return res


# speedup vs baseline: 1.2394x; 1.0694x over previous
import functools

import jax
import jax.numpy as jnp
from jax import lax
from jax.experimental import pallas as pl
from jax.experimental.pallas import tpu as pltpu
from jax.experimental.pallas import tpu_sc as plsc

N_HOPS = 2

_NC = 2
_NS = 16
_NW = _NC * _NS
_B = 80


def _sc_compiler_params():
    import dataclasses
    cp = pltpu.CompilerParams()
    if "needs_layout_passes" in pltpu.CompilerParams.__dataclass_fields__:
        cp = dataclasses.replace(cp, needs_layout_passes=False)
    return cp


def _row_split(n):
    base = (n // _NS) // 8 * 8
    last = n - base * (_NS - 1)
    assert last % 8 == 0
    return base, last


def _score_body(nb, n, d, emb_hbm, rel_hbm, pack_hbm, score_out, tilemax_out,
                rel_v, ib_v, sblk_v, maxtab_v, hrows_v, trows_v, semh, semt):
    c = lax.axis_index("c")
    s = lax.axis_index("s")
    wid = c * _NS + s

    pltpu.sync_copy(rel_hbm, rel_v)

    neg = jnp.full((16,), -3.0e38, jnp.float32)

    @pl.loop(0, n, step=16)
    def _(i):
        maxtab_v[pl.ds(i, 16)] = neg

    def issue(p, blk):
        pltpu.sync_copy(pack_hbm.at[wid, blk], ib_v[p])
        pltpu.async_copy(emb_hbm.at[ib_v[p].at[0]], hrows_v[p], semh[p])
        pltpu.async_copy(emb_hbm.at[ib_v[p].at[1]], trows_v[p], semt[p])

    def wait_gather(p):
        pltpu.make_async_copy(emb_hbm.at[ib_v[p].at[0]], hrows_v[p],
                              semh[p]).wait()
        pltpu.make_async_copy(emb_hbm.at[ib_v[p].at[1]], trows_v[p],
                              semt[p]).wait()

    def compute(p, blk):
        @pl.loop(0, _B, step=16)
        def _(eg):
            eids = lax.iota(jnp.int32, 16) + eg
            tys = ib_v[p][2, pl.ds(eg, 16)]

            acc = jnp.zeros((16,), jnp.float32)
            for i in range(d):
                dsp = jnp.zeros((16,), jnp.int32) + i
                h = plsc.load_gather(hrows_v[p], [eids, dsp])
                t = plsc.load_gather(trows_v[p], [eids, dsp])
                r = plsc.load_gather(rel_v, [tys, dsp])
                acc = acc + h * t * r
            sblk_v[blk, pl.ds(eg, 16)] = acc

            hh = ib_v[p][0, pl.ds(eg, 16)]

            def mcond(pend):
                return jnp.any(pend)

            def mbody(pend):
                cur = plsc.load_gather(maxtab_v, [hh])
                need = pend & (acc > cur)
                plsc.store_scatter(maxtab_v, [hh], acc, mask=need)
                cur2 = plsc.load_gather(maxtab_v, [hh])
                return need & (cur2 < acc)

            lax.while_loop(mcond, mbody, jnp.ones((16,), jnp.bool_))

    issue(0, 0)

    @pl.loop(0, nb, step=2)
    def _(blk):
        @pl.when(blk + 1 < nb)
        def _():
            issue(1, blk + 1)

        wait_gather(0)
        compute(0, blk)

        @pl.when(blk + 2 < nb)
        def _():
            issue(0, blk + 2)

        @pl.when(blk + 1 < nb)
        def _():
            wait_gather(1)
            compute(1, blk + 1)

    pltpu.sync_copy(sblk_v, score_out.at[wid])
    pltpu.sync_copy(maxtab_v, tilemax_out.at[pl.ds(wid * n, n)])


def _shift_body(nb, n, head_hbm, score_hbm, tilemax_hbm, shifted_out,
                head_v, sblk_v, segmax_v, m0_v, m1_v, smax_sh):
    c = lax.axis_index("c")
    s = lax.axis_index("s")
    wid = c * _NS + s
    rbase, rlast = _row_split(n)

    pltpu.sync_copy(head_hbm.at[wid], head_v)
    pltpu.sync_copy(score_hbm.at[wid], sblk_v)

    def _combine_rows(r0, cnt):
        pltpu.sync_copy(tilemax_hbm.at[pl.ds(r0, cnt)], m0_v.at[pl.ds(0, cnt)])

        @pl.loop(1, _NW)
        def _(j):
            pltpu.sync_copy(tilemax_hbm.at[pl.ds(j * n + r0, cnt)],
                            m1_v.at[pl.ds(0, cnt)])

            @pl.loop(0, cnt, step=16)
            def _(i):
                m0_v[pl.ds(i, 16)] = jnp.maximum(m0_v[pl.ds(i, 16)],
                                                 m1_v[pl.ds(i, 16)])

        pltpu.sync_copy(m0_v.at[pl.ds(0, cnt)], smax_sh.at[pl.ds(r0, cnt)])

    @pl.when(s < _NS - 1)
    def _():
        _combine_rows(s * rbase, rbase)

    @pl.when(s == _NS - 1)
    def _():
        _combine_rows((_NS - 1) * rbase, rlast)

    plsc.subcore_barrier()
    pltpu.sync_copy(smax_sh, segmax_v)

    @pl.loop(0, nb)
    def _(blk):
        @pl.loop(0, _B, step=16)
        def _(eg):
            hh = head_v[blk, pl.ds(eg, 16)]
            mx = plsc.load_gather(segmax_v, [hh])
            sblk_v[blk, pl.ds(eg, 16)] = sblk_v[blk, pl.ds(eg, 16)] - mx

    pltpu.sync_copy(sblk_v, shifted_out.at[wid])


def _hop_body(nb, n, d, x_hbm, rel_hbm, pack_hbm, zeros_hbm, part_out,
              rel_v, ib_v, hsc_v, rows_v, orows_v, acc_sh, semg, sems):
    c = lax.axis_index("c")
    s = lax.axis_index("s")
    wid = c * _NS + s
    rbase, rlast = _row_split(n)

    pltpu.sync_copy(rel_hbm, rel_v)

    @pl.when(s < _NS - 1)
    def _():
        pltpu.sync_copy(zeros_hbm.at[pl.ds(s * rbase, rbase)],
                        acc_sh.at[pl.ds(s * rbase, rbase)])

    @pl.when(s == _NS - 1)
    def _():
        pltpu.sync_copy(zeros_hbm.at[pl.ds((_NS - 1) * rbase, rlast)],
                        acc_sh.at[pl.ds((_NS - 1) * rbase, rlast)])

    plsc.subcore_barrier()

    def issue(p, blk):
        pltpu.sync_copy(pack_hbm.at[wid, blk], ib_v[p])
        pltpu.async_copy(x_hbm.at[ib_v[p].at[1]], rows_v[p], semg[p])

    def compute(p, blk):
        @pl.loop(0, _B, step=16)
        def _(eg):
            eids = lax.iota(jnp.int32, 16) + eg
            sc = plsc.bitcast(ib_v[p][3, pl.ds(eg, 16)], jnp.float32)
            w = jnp.exp(sc)
            tys = ib_v[p][2, pl.ds(eg, 16)]
            hsc_v[p][pl.ds(eg, 16)] = ib_v[p][0, pl.ds(eg, 16)]

            for i in range(d):
                dsp = jnp.zeros((16,), jnp.int32) + i
                xv = plsc.load_gather(rows_v[p], [eids, dsp])
                rv = plsc.load_gather(rel_v, [tys, dsp])
                plsc.store_scatter(orows_v[p], [eids, dsp], xv * rv * w)

    def scatter(p):
        pltpu.async_copy(orows_v[p], acc_sh.at[hsc_v[p]], sems[p], add=True)

    def wait_scatter(p):
        pltpu.make_async_copy(orows_v[p], acc_sh.at[hsc_v[p]],
                              sems[p]).wait()

    issue(0, 0)

    @pl.loop(0, nb, step=2)
    def _(blk):
        @pl.when(blk + 1 < nb)
        def _():
            issue(1, blk + 1)

        pltpu.make_async_copy(x_hbm.at[ib_v[0].at[1]], rows_v[0],
                              semg[0]).wait()

        @pl.when(blk >= 2)
        def _():
            wait_scatter(0)

        compute(0, blk)
        scatter(0)

        @pl.when(blk + 2 < nb)
        def _():
            issue(0, blk + 2)

        @pl.when(blk + 1 < nb)
        def _():
            pltpu.make_async_copy(x_hbm.at[ib_v[1].at[1]], rows_v[1],
                                  semg[1]).wait()

            @pl.when(blk >= 2)
            def _():
                wait_scatter(1)

            compute(1, blk + 1)
            scatter(1)

    wait_scatter(0)
    wait_scatter(1)
    plsc.subcore_barrier()

    @pl.when(s < _NS - 1)
    def _():
        pltpu.sync_copy(acc_sh.at[pl.ds(s * rbase, rbase)],
                        part_out.at[c, pl.ds(s * rbase, rbase)])

    @pl.when(s == _NS - 1)
    def _():
        pltpu.sync_copy(acc_sh.at[pl.ds((_NS - 1) * rbase, rlast)],
                        part_out.at[c, pl.ds((_NS - 1) * rbase, rlast)])


def _combine_body(p_ref, res_ref, xout_ref, resout_ref):
    sblk = p_ref[0] + p_ref[1]
    nrm = jnp.sqrt(jnp.sum(sblk * sblk, axis=1, keepdims=True))
    x = sblk / jnp.maximum(nrm, 1e-12)
    xout_ref[...] = x
    resout_ref[...] = res_ref[...] + x


def kernel(entity_emb, relation_emb, edge_index, edge_type):
    n, d = entity_emb.shape
    e = edge_index.shape[1]
    nrel = relation_emb.shape[0]
    epw = e // _NW
    nb = epw // _B
    assert epw * _NW == e and nb * _B == epw

    head = edge_index[0].astype(jnp.int32)
    tail = edge_index[1].astype(jnp.int32)
    ty = edge_type.astype(jnp.int32) - 1
    head1, tail1, ty1 = (a.reshape(_NW, nb, _B) for a in (head, tail, ty))
    rel = relation_emb.astype(jnp.float32)
    zeros = jnp.zeros((n, d), jnp.float32)
    pack0 = jnp.stack([head1, tail1, ty1], axis=2)

    mesh = plsc.VectorSubcoreMesh(core_axis_name="c", subcore_axis_name="s")
    cp = _sc_compiler_params()

    score_kernel = pl.kernel(
        functools.partial(_score_body, nb, n, d),
        out_type=(jax.ShapeDtypeStruct((_NW, nb, _B), jnp.float32),
                  jax.ShapeDtypeStruct((_NW * n,), jnp.float32)),
        mesh=mesh,
        compiler_params=cp,
        scratch_types=[
            pltpu.VMEM((nrel, d), jnp.float32),
            [pltpu.VMEM((3, _B), jnp.int32)] * 2,
            pltpu.VMEM((nb, _B), jnp.float32),
            pltpu.VMEM((n,), jnp.float32),
            [pltpu.VMEM((_B, d), jnp.float32)] * 2,
            [pltpu.VMEM((_B, d), jnp.float32)] * 2,
            [pltpu.SemaphoreType.DMA] * 2,
            [pltpu.SemaphoreType.DMA] * 2,
        ],
    )

    shift_kernel = pl.kernel(
        functools.partial(_shift_body, nb, n),
        out_type=jax.ShapeDtypeStruct((_NW, nb, _B), jnp.float32),
        mesh=mesh,
        compiler_params=cp,
        scratch_types=[
            pltpu.VMEM((nb, _B), jnp.int32),
            pltpu.VMEM((nb, _B), jnp.float32),
            pltpu.VMEM((n,), jnp.float32),
            pltpu.VMEM((_row_split(n)[1],), jnp.float32),
            pltpu.VMEM((_row_split(n)[1],), jnp.float32),
            pltpu.VMEM_SHARED((n,), jnp.float32),
        ],
    )

    hop_kernel = pl.kernel(
        functools.partial(_hop_body, nb, n, d),
        out_type=jax.ShapeDtypeStruct((_NC, n, d), jnp.float32),
        mesh=mesh,
        compiler_params=cp,
        scratch_types=[
            pltpu.VMEM((nrel, d), jnp.float32),
            [pltpu.VMEM((4, _B), jnp.int32)] * 2,
            [pltpu.VMEM((_B,), jnp.int32)] * 2,
            [pltpu.VMEM((_B, d), jnp.float32)] * 2,
            [pltpu.VMEM((_B, d), jnp.float32)] * 2,
            pltpu.VMEM_SHARED((n, d), jnp.float32),
            [pltpu.SemaphoreType.DMA] * 2,
            [pltpu.SemaphoreType.DMA] * 2,
        ],
    )

    rows_blk = 1000
    combine = pl.pallas_call(
        _combine_body,
        grid=(n // rows_blk,),
        in_specs=[
            pl.BlockSpec((_NC, rows_blk, d), lambda i: (0, i, 0)),
            pl.BlockSpec((rows_blk, d), lambda i: (i, 0)),
        ],
        out_specs=[
            pl.BlockSpec((rows_blk, d), lambda i: (i, 0)),
            pl.BlockSpec((rows_blk, d), lambda i: (i, 0)),
        ],
        out_shape=(jax.ShapeDtypeStruct((n, d), jnp.float32),
                   jax.ShapeDtypeStruct((n, d), jnp.float32)),
    )

    score, tilemax = score_kernel(entity_emb, rel, pack0)
    shifted = shift_kernel(head1, score, tilemax)
    pack = jnp.stack(
        [head1, tail1, ty1,
         jax.lax.bitcast_convert_type(shifted, jnp.int32)], axis=2)
    res = entity_emb
    x = entity_emb
    for _ in range(N_HOPS):
        part = hop_kernel(x, rel, pack, zeros)
        x, res = combine(part, res)
    return res


# speedup vs baseline: 4.5444x; 3.6667x over previous
import functools

import jax
import jax.numpy as jnp
from jax import lax
from jax.experimental import pallas as pl
from jax.experimental.pallas import tpu as pltpu
from jax.experimental.pallas import tpu_sc as plsc

N_HOPS = 2

_NC = 2
_NS = 16
_NW = _NC * _NS
_B = 80


def _sc_compiler_params():
    import dataclasses
    cp = pltpu.CompilerParams()
    if "needs_layout_passes" in pltpu.CompilerParams.__dataclass_fields__:
        cp = dataclasses.replace(cp, needs_layout_passes=False)
    return cp


def _row_split(n):
    base = (n // _NS) // 8 * 8
    last = n - base * (_NS - 1)
    assert last % 8 == 0
    return base, last


def _score_body(nb, n, d, emb_hbm, rel_hbm, ht_hbm,
                score_out, tilemax_out,
                rel_v, ht_v, sblk_v, maxtab_v, hrows_v, trows_v,
                semh, semt):
    c = lax.axis_index("c")
    s = lax.axis_index("s")
    wid = c * _NS + s
    epw = nb * _B

    pltpu.sync_copy(rel_hbm, rel_v)

    neg = jnp.full((16,), -3.0e38, jnp.float32)

    @pl.loop(0, n, step=16)
    def _(i):
        maxtab_v[pl.ds(i, 16)] = neg

    def issue(p, blk):
        pltpu.sync_copy(ht_hbm.at[wid * nb + blk], ht_v[p])
        pltpu.async_copy(emb_hbm.at[ht_v[p].at[0]], hrows_v[p], semh[p])
        pltpu.async_copy(emb_hbm.at[ht_v[p].at[1]], trows_v[p], semt[p])

    def wait_gather(p):
        pltpu.make_async_copy(emb_hbm.at[ht_v[p].at[0]], hrows_v[p],
                              semh[p]).wait()
        pltpu.make_async_copy(emb_hbm.at[ht_v[p].at[1]], trows_v[p],
                              semt[p]).wait()

    def compute(p, blk):
        @pl.loop(0, _B, step=16)
        def _(eg):
            eids = lax.iota(jnp.int32, 16) + eg
            tys = ht_v[p][2, pl.ds(eg, 16)]
            sums = jnp.zeros((16,), jnp.float32)
            for k in range(16):
                e = eg + k
                ty_e = tys[k]
                acc = jnp.zeros((16,), jnp.float32)
                for dc in range(0, d, 16):
                    h = hrows_v[p][e, pl.ds(dc, 16)]
                    t = trows_v[p][e, pl.ds(dc, 16)]
                    r = rel_v[ty_e, pl.ds(dc, 16)]
                    acc = acc + h * t * r
                sums = jnp.where(eids == e, jnp.sum(acc), sums)
            sblk_v[pl.ds(blk * _B + eg, 16)] = sums

            hh = ht_v[p][0, pl.ds(eg, 16)]

            def mcond(pend):
                return jnp.any(pend)

            def mbody(pend):
                cur = plsc.load_gather(maxtab_v, [hh])
                need = pend & (sums > cur)
                plsc.store_scatter(maxtab_v, [hh], sums, mask=need)
                cur2 = plsc.load_gather(maxtab_v, [hh])
                return need & (cur2 < sums)

            lax.while_loop(mcond, mbody, jnp.ones((16,), jnp.bool_))

    issue(0, 0)

    @pl.loop(0, nb, step=2)
    def _(blk):
        @pl.when(blk + 1 < nb)
        def _():
            issue(1, blk + 1)

        wait_gather(0)
        compute(0, blk)

        @pl.when(blk + 2 < nb)
        def _():
            issue(0, blk + 2)

        @pl.when(blk + 1 < nb)
        def _():
            wait_gather(1)
            compute(1, blk + 1)

    pltpu.sync_copy(sblk_v, score_out.at[pl.ds(wid * epw, epw)])
    pltpu.sync_copy(maxtab_v, tilemax_out.at[pl.ds(wid * n, n)])


def _shift_body(nb, n, head_hbm, score_hbm, tilemax_hbm, w_out,
                head_v, sblk_v, segmax_v, m0_v, m1_v, smax_sh):
    c = lax.axis_index("c")
    s = lax.axis_index("s")
    wid = c * _NS + s
    epw = nb * _B
    rbase, rlast = _row_split(n)

    pltpu.sync_copy(head_hbm.at[pl.ds(wid * epw, epw)], head_v)
    pltpu.sync_copy(score_hbm.at[pl.ds(wid * epw, epw)], sblk_v)

    def _combine_rows(r0, cnt):
        pltpu.sync_copy(tilemax_hbm.at[pl.ds(r0, cnt)], m0_v.at[pl.ds(0, cnt)])

        @pl.loop(1, _NW)
        def _(j):
            pltpu.sync_copy(tilemax_hbm.at[pl.ds(j * n + r0, cnt)],
                            m1_v.at[pl.ds(0, cnt)])

            @pl.loop(0, cnt, step=16)
            def _(i):
                m0_v[pl.ds(i, 16)] = jnp.maximum(m0_v[pl.ds(i, 16)],
                                                 m1_v[pl.ds(i, 16)])

        pltpu.sync_copy(m0_v.at[pl.ds(0, cnt)], smax_sh.at[pl.ds(r0, cnt)])

    @pl.when(s < _NS - 1)
    def _():
        _combine_rows(s * rbase, rbase)

    @pl.when(s == _NS - 1)
    def _():
        _combine_rows((_NS - 1) * rbase, rlast)

    plsc.subcore_barrier()
    pltpu.sync_copy(smax_sh, segmax_v)

    @pl.loop(0, epw, step=16)
    def _(i):
        hh = head_v[pl.ds(i, 16)]
        mx = plsc.load_gather(segmax_v, [hh])
        sblk_v[pl.ds(i, 16)] = jnp.exp(sblk_v[pl.ds(i, 16)] - mx)

    pltpu.sync_copy(sblk_v, w_out.at[pl.ds(wid * epw, epw)])


def _hop_body(nb, n, d, x_hbm, rel_hbm, ht_hbm, zeros_hbm, part_out,
              rel_v, ht_v, hsc_v, rows_v, orows_v, acc_sh,
              semg, sems):
    c = lax.axis_index("c")
    s = lax.axis_index("s")
    wid = c * _NS + s
    rbase, rlast = _row_split(n)

    pltpu.sync_copy(rel_hbm, rel_v)

    @pl.when(s < _NS - 1)
    def _():
        pltpu.sync_copy(zeros_hbm.at[pl.ds(s * rbase, rbase)],
                        acc_sh.at[pl.ds(s * rbase, rbase)])

    @pl.when(s == _NS - 1)
    def _():
        pltpu.sync_copy(zeros_hbm.at[pl.ds((_NS - 1) * rbase, rlast)],
                        acc_sh.at[pl.ds((_NS - 1) * rbase, rlast)])

    plsc.subcore_barrier()

    def issue(p, blk):
        pltpu.sync_copy(ht_hbm.at[wid * nb + blk], ht_v[p])
        pltpu.async_copy(x_hbm.at[ht_v[p].at[1]], rows_v[p], semg[p])

    def compute(p, blk):
        @pl.loop(0, _B, step=16)
        def _(eg):
            hsc_v[p][pl.ds(eg, 16)] = ht_v[p][0, pl.ds(eg, 16)]
            tys = ht_v[p][2, pl.ds(eg, 16)]
            ws = plsc.bitcast(ht_v[p][3, pl.ds(eg, 16)], jnp.float32)
            for k in range(16):
                e = eg + k
                ty_e = tys[k]
                w16 = jnp.zeros((16,), jnp.float32) + ws[k]
                for dc in range(0, d, 16):
                    xv = rows_v[p][e, pl.ds(dc, 16)]
                    rv = rel_v[ty_e, pl.ds(dc, 16)]
                    orows_v[p][e, pl.ds(dc, 16)] = xv * rv * w16

    def scatter(p):
        pltpu.async_copy(orows_v[p], acc_sh.at[hsc_v[p]], sems[p], add=True)

    def wait_scatter(p):
        pltpu.make_async_copy(orows_v[p], acc_sh.at[hsc_v[p]],
                              sems[p]).wait()

    issue(0, 0)

    @pl.loop(0, nb, step=2)
    def _(blk):
        @pl.when(blk + 1 < nb)
        def _():
            issue(1, blk + 1)

        pltpu.make_async_copy(x_hbm.at[ht_v[0].at[1]], rows_v[0],
                              semg[0]).wait()

        @pl.when(blk >= 2)
        def _():
            wait_scatter(0)

        compute(0, blk)
        scatter(0)

        @pl.when(blk + 2 < nb)
        def _():
            issue(0, blk + 2)

        @pl.when(blk + 1 < nb)
        def _():
            pltpu.make_async_copy(x_hbm.at[ht_v[1].at[1]], rows_v[1],
                                  semg[1]).wait()

            @pl.when(blk >= 2)
            def _():
                wait_scatter(1)

            compute(1, blk + 1)
            scatter(1)

    wait_scatter(0)
    wait_scatter(1)
    plsc.subcore_barrier()

    @pl.when(s < _NS - 1)
    def _():
        pltpu.sync_copy(acc_sh.at[pl.ds(s * rbase, rbase)],
                        part_out.at[c, pl.ds(s * rbase, rbase)])

    @pl.when(s == _NS - 1)
    def _():
        pltpu.sync_copy(acc_sh.at[pl.ds((_NS - 1) * rbase, rlast)],
                        part_out.at[c, pl.ds((_NS - 1) * rbase, rlast)])


def _combine_body(p_ref, res_ref, xout_ref, resout_ref):
    sblk = p_ref[0] + p_ref[1]
    nrm = jnp.sqrt(jnp.sum(sblk * sblk, axis=1, keepdims=True))
    x = sblk / jnp.maximum(nrm, 1e-12)
    xout_ref[...] = x
    resout_ref[...] = res_ref[...] + x


def kernel(entity_emb, relation_emb, edge_index, edge_type):
    n, d = entity_emb.shape
    e = edge_index.shape[1]
    nrel = relation_emb.shape[0]
    epw = e // _NW
    nb = epw // _B
    assert epw * _NW == e and nb * _B == epw

    head = edge_index[0].astype(jnp.int32)
    tail = edge_index[1].astype(jnp.int32)
    ty = edge_type.astype(jnp.int32) - 1
    rel = relation_emb.astype(jnp.float32)
    zeros = jnp.zeros((n, d), jnp.float32)
    ht = jnp.stack([head.reshape(_NW * nb, _B),
                    tail.reshape(_NW * nb, _B),
                    ty.reshape(_NW * nb, _B)], axis=1)

    mesh = plsc.VectorSubcoreMesh(core_axis_name="c", subcore_axis_name="s")
    cp = _sc_compiler_params()

    score_kernel = pl.kernel(
        functools.partial(_score_body, nb, n, d),
        out_type=(jax.ShapeDtypeStruct((e,), jnp.float32),
                  jax.ShapeDtypeStruct((_NW * n,), jnp.float32)),
        mesh=mesh,
        compiler_params=cp,
        scratch_types=[
            pltpu.VMEM((nrel, d), jnp.float32),
            [pltpu.VMEM((3, _B), jnp.int32)] * 2,
            pltpu.VMEM((nb * _B,), jnp.float32),
            pltpu.VMEM((n,), jnp.float32),
            [pltpu.VMEM((_B, d), jnp.float32)] * 2,
            [pltpu.VMEM((_B, d), jnp.float32)] * 2,
            [pltpu.SemaphoreType.DMA] * 2,
            [pltpu.SemaphoreType.DMA] * 2,
        ],
    )

    shift_kernel = pl.kernel(
        functools.partial(_shift_body, nb, n),
        out_type=jax.ShapeDtypeStruct((e,), jnp.float32),
        mesh=mesh,
        compiler_params=cp,
        scratch_types=[
            pltpu.VMEM((nb * _B,), jnp.int32),
            pltpu.VMEM((nb * _B,), jnp.float32),
            pltpu.VMEM((n,), jnp.float32),
            pltpu.VMEM((_row_split(n)[1],), jnp.float32),
            pltpu.VMEM((_row_split(n)[1],), jnp.float32),
            pltpu.VMEM_SHARED((n,), jnp.float32),
        ],
    )

    hop_kernel = pl.kernel(
        functools.partial(_hop_body, nb, n, d),
        out_type=jax.ShapeDtypeStruct((_NC, n, d), jnp.float32),
        mesh=mesh,
        compiler_params=cp,
        scratch_types=[
            pltpu.VMEM((nrel, d), jnp.float32),
            [pltpu.VMEM((4, _B), jnp.int32)] * 2,
            [pltpu.VMEM((_B,), jnp.int32)] * 2,
            [pltpu.VMEM((_B, d), jnp.float32)] * 2,
            [pltpu.VMEM((_B, d), jnp.float32)] * 2,
            pltpu.VMEM_SHARED((n, d), jnp.float32),
            [pltpu.SemaphoreType.DMA] * 2,
            [pltpu.SemaphoreType.DMA] * 2,
        ],
    )

    rows_blk = 1000
    combine = pl.pallas_call(
        _combine_body,
        grid=(n // rows_blk,),
        in_specs=[
            pl.BlockSpec((_NC, rows_blk, d), lambda i: (0, i, 0)),
            pl.BlockSpec((rows_blk, d), lambda i: (i, 0)),
        ],
        out_specs=[
            pl.BlockSpec((rows_blk, d), lambda i: (i, 0)),
            pl.BlockSpec((rows_blk, d), lambda i: (i, 0)),
        ],
        out_shape=(jax.ShapeDtypeStruct((n, d), jnp.float32),
                   jax.ShapeDtypeStruct((n, d), jnp.float32)),
    )

    score, tilemax = score_kernel(entity_emb, rel, ht)
    w = shift_kernel(head, score, tilemax)
    htw = jnp.concatenate(
        [ht, jax.lax.bitcast_convert_type(w, jnp.int32)
             .reshape(_NW * nb, 1, _B)], axis=1)
    res = entity_emb
    x = entity_emb
    for _ in range(N_HOPS):
        part = hop_kernel(x, rel, htw, zeros)
        x, res = combine(part, res)
    return res
